# baseline (speedup 1.0000x reference)
import dataclasses
import functools

import jax
import jax.numpy as jnp
from jax import lax
from jax.experimental import pallas as pl
from jax.experimental.pallas import tpu as pltpu
from jax.experimental.pallas import tpu_sc as plsc

C = 64
D = 512
NSUP = 8192
NQ = 16384
NC = 2
NT = 16
NW = NC * NT
ROWS_PER_TILE = NSUP // NW
CHUNK = 64
NCHUNK = ROWS_PER_TILE // CHUNK
QBLK = 4096


def _segment_sums_sc(x_flat, y_support):
    mesh = plsc.VectorSubcoreMesh(core_axis_name="c", subcore_axis_name="s")
    cp = pltpu.CompilerParams()
    if "needs_layout_passes" in pltpu.CompilerParams.__dataclass_fields__:
        cp = dataclasses.replace(cp, needs_layout_passes=False)

    @functools.partial(
        pl.kernel,
        out_type=(
            jax.ShapeDtypeStruct((NW, C, D), jnp.float32),
            jax.ShapeDtypeStruct((NW, C), jnp.float32),
        ),
        mesh=mesh,
        scratch_types=[
            pltpu.VMEM((2, CHUNK, D), jnp.float32),
            pltpu.VMEM((2, CHUNK), jnp.int32),
            pltpu.VMEM((C, D), jnp.float32),
            pltpu.VMEM((C,), jnp.float32),
            pltpu.SemaphoreType.DMA,
            pltpu.SemaphoreType.DMA,
            pltpu.SemaphoreType.DMA,
            pltpu.SemaphoreType.DMA,
        ],
        compiler_params=cp,
    )
    def seg_kernel(x_hbm, y_hbm, sums_hbm, cnt_hbm, rows_v, idx_v, acc_v,
                   hist_v, semr0, semr1, semi0, semi1):
        cid = lax.axis_index("c")
        sid = lax.axis_index("s")
        wid = cid * NT + sid
        semr = (semr0, semr1)
        semi = (semi0, semi1)

        def start_fetch(k):
            b = k % 2
            off = wid * ROWS_PER_TILE + k * CHUNK
            hr = pltpu.async_copy(
                x_hbm.at[pl.ds(off, CHUNK)], rows_v.at[b], semr[b])
            hi = pltpu.async_copy(
                y_hbm.at[pl.ds(off, CHUNK)], idx_v.at[b], semi[b])
            return hr, hi

        inflight = start_fetch(0)

        zero16 = jnp.zeros((16,), jnp.float32)

        @pl.loop(0, C)
        def _(i):
            for u in range(D // 16):
                acc_v[i, pl.ds(u * 16, 16)] = zero16

        for u in range(C // 16):
            hist_v[pl.ds(u * 16, 16)] = zero16

        col0 = lax.broadcasted_iota(jnp.int32, (16,), 0)
        lane0 = col0 == 0
        ones16 = jnp.full((16,), 1.0, jnp.float32)

        for k in range(NCHUNK):
            b = k % 2
            hr, hi = inflight
            hr.wait()
            hi.wait()
            if k + 1 < NCHUNK:
                inflight = start_fetch(k + 1)

            @plsc.parallel_loop(0, CHUNK, unroll=16)
            def _(r):
                lbl = plsc.load_gather(
                    idx_v.at[b], [jnp.zeros((16,), jnp.int32) + r])
                plsc.addupdate_scatter(hist_v, [lbl], ones16, mask=lane0)
                for j in range(D // 16):
                    data = rows_v[b, r, pl.ds(j * 16, 16)]
                    plsc.addupdate_scatter(
                        acc_v, [lbl, col0 + j * 16], data)

        pltpu.sync_copy(acc_v, sums_hbm.at[wid])
        pltpu.sync_copy(hist_v, cnt_hbm.at[wid])

    return seg_kernel(x_flat, y_support)


DA = D + 128


def _prep_queries_tc(x_query):
    def body(q_ref, a_ref):
        q = q_ref[...]
        qn = jnp.sum(q * q, axis=1, keepdims=True)
        qn_hi = qn.astype(jnp.bfloat16)
        qn_lo = (qn - qn_hi.astype(jnp.float32)).astype(jnp.bfloat16)
        lane = lax.broadcasted_iota(jnp.int32, (QBLK, 128), 1)
        tail = jnp.where(lane == 0, qn_hi.astype(jnp.float32),
                         jnp.where(lane == 1, qn_lo.astype(jnp.float32), 0.0))
        a_ref[...] = jnp.concatenate(
            [q.astype(jnp.bfloat16), tail.astype(jnp.bfloat16)], axis=1)

    return pl.pallas_call(
        body,
        grid=(NQ // QBLK,),
        in_specs=[pl.BlockSpec((QBLK, D), lambda i: (i, 0))],
        out_specs=pl.BlockSpec((QBLK, DA), lambda i: (i, 0)),
        out_shape=jax.ShapeDtypeStruct((NQ, DA), jnp.bfloat16),
    )(x_query)


def _distances_tc(partial_sums, partial_counts, q_aug):
    def body(ps_ref, pc_ref, a_ref, out_ref, r_ref, pn_ref):
        @pl.when(pl.program_id(0) == 0)
        def _():
            sums = ps_ref[0]
            for w in range(1, NW):
                sums = sums + ps_ref[w]
            counts = jnp.sum(pc_ref[...], axis=0)
            proto = sums * (1.0 / counts)[:, None]
            pn_ref[...] = jnp.sum(proto * proto, axis=1)[None, :]
            lane = lax.broadcasted_iota(jnp.int32, (C, 128), 1)
            tail = jnp.where(lane < 2, -0.5, 0.0)
            r_ref[...] = jnp.concatenate(
                [proto.astype(jnp.bfloat16), tail.astype(jnp.bfloat16)],
                axis=1)

        dot = lax.dot_general(
            a_ref[...], r_ref[...], (((1,), (1,)), ((), ())),
            preferred_element_type=jnp.float32,
        )
        out_ref[...] = pn_ref[...] - 2.0 * dot

    return pl.pallas_call(
        body,
        grid=(NQ // QBLK,),
        in_specs=[
            pl.BlockSpec((NW, C, D), lambda i: (0, 0, 0)),
            pl.BlockSpec((NW, C), lambda i: (0, 0)),
            pl.BlockSpec((QBLK, DA), lambda i: (i, 0)),
        ],
        out_specs=pl.BlockSpec((QBLK, C), lambda i: (i, 0)),
        out_shape=jax.ShapeDtypeStruct((NQ, C), jnp.float32),
        scratch_shapes=[
            pltpu.VMEM((C, DA), jnp.bfloat16),
            pltpu.VMEM((1, C), jnp.float32),
        ],
    )(partial_sums, partial_counts, q_aug)


def kernel(x_support, y_support, x_query):
    q_aug = _prep_queries_tc(x_query)
    sums, counts = _segment_sums_sc(
        x_support, y_support.astype(jnp.int32))
    return _distances_tc(sums, counts, q_aug)


# speedup vs baseline: 1.0485x; 1.0485x over previous
import dataclasses
import functools

import jax
import jax.numpy as jnp
from jax import lax
from jax.experimental import pallas as pl
from jax.experimental.pallas import tpu as pltpu
from jax.experimental.pallas import tpu_sc as plsc

C = 64
D = 512
NSUP = 8192
NQ = 16384
NC = 2
NT = 16
NW = NC * NT
ROWS_PER_TILE = NSUP // NW
CHUNK = 64
NCHUNK = ROWS_PER_TILE // CHUNK
QBLK = 4096


def _segment_sums_sc(x_flat, y_support):
    mesh = plsc.VectorSubcoreMesh(core_axis_name="c", subcore_axis_name="s")
    cp = pltpu.CompilerParams()
    if "needs_layout_passes" in pltpu.CompilerParams.__dataclass_fields__:
        cp = dataclasses.replace(cp, needs_layout_passes=False)

    @functools.partial(
        pl.kernel,
        out_type=(
            jax.ShapeDtypeStruct((NW, C * D), jnp.float32),
            jax.ShapeDtypeStruct((NW, C), jnp.float32),
        ),
        mesh=mesh,
        scratch_types=[
            pltpu.VMEM((2, CHUNK, D), jnp.float32),
            pltpu.VMEM((2, CHUNK), jnp.int32),
            pltpu.VMEM((1, C * D), jnp.float32),
            pltpu.VMEM((C,), jnp.float32),
            pltpu.SemaphoreType.DMA,
            pltpu.SemaphoreType.DMA,
            pltpu.SemaphoreType.DMA,
            pltpu.SemaphoreType.DMA,
        ],
        compiler_params=cp,
    )
    def seg_kernel(x_hbm, y_hbm, sums_hbm, cnt_hbm, rows_v, idx_v, acc_v,
                   hist_v, semr0, semr1, semi0, semi1):
        cid = lax.axis_index("c")
        sid = lax.axis_index("s")
        wid = cid * NT + sid
        semr = (semr0, semr1)
        semi = (semi0, semi1)

        def start_fetch(k):
            b = k % 2
            off = wid * ROWS_PER_TILE + k * CHUNK
            hr = pltpu.async_copy(
                x_hbm.at[pl.ds(off, CHUNK)], rows_v.at[b], semr[b])
            hi = pltpu.async_copy(
                y_hbm.at[pl.ds(off, CHUNK)], idx_v.at[b], semi[b])
            return hr, hi

        inflight = start_fetch(0)

        zero16 = jnp.zeros((16,), jnp.float32)

        @pl.loop(0, C * D, step=128)
        def _(i):
            for u in range(8):
                acc_v[0, pl.ds(i + u * 16, 16)] = zero16

        for u in range(C // 16):
            hist_v[pl.ds(u * 16, 16)] = zero16

        col0 = lax.broadcasted_iota(jnp.int32, (16,), 0)
        lane0 = col0 == 0
        ones16 = jnp.full((16,), 1.0, jnp.float32)
        zeroi16 = jnp.zeros((16,), jnp.int32)

        for k in range(NCHUNK):
            b = k % 2
            hr, hi = inflight
            hr.wait()
            hi.wait()
            if k + 1 < NCHUNK:
                inflight = start_fetch(k + 1)

            @plsc.parallel_loop(0, CHUNK, unroll=8)
            def _(r):
                lbl = plsc.load_gather(idx_v.at[b], [zeroi16 + r])
                plsc.addupdate_scatter(hist_v, [lbl], ones16, mask=lane0)
                base = lbl * D + col0
                for j in range(D // 16):
                    data = rows_v[b, r, pl.ds(j * 16, 16)]
                    plsc.addupdate_scatter(
                        acc_v, [zeroi16, base + j * 16], data)

        pltpu.sync_copy(acc_v.at[0], sums_hbm.at[wid])
        pltpu.sync_copy(hist_v, cnt_hbm.at[wid])

    return seg_kernel(x_flat, y_support)


DA = D + 128


def _prep_queries_tc(x_query):
    def body(q_ref, a_ref):
        q = q_ref[...]
        qn = jnp.sum(q * q, axis=1, keepdims=True)
        qn_hi = qn.astype(jnp.bfloat16)
        qn_lo = (qn - qn_hi.astype(jnp.float32)).astype(jnp.bfloat16)
        lane = lax.broadcasted_iota(jnp.int32, (QBLK, 128), 1)
        tail = jnp.where(lane == 0, qn_hi.astype(jnp.float32),
                         jnp.where(lane == 1, qn_lo.astype(jnp.float32), 0.0))
        a_ref[...] = jnp.concatenate(
            [q.astype(jnp.bfloat16), tail.astype(jnp.bfloat16)], axis=1)

    return pl.pallas_call(
        body,
        grid=(NQ // QBLK,),
        in_specs=[pl.BlockSpec((QBLK, D), lambda i: (i, 0))],
        out_specs=pl.BlockSpec((QBLK, DA), lambda i: (i, 0)),
        out_shape=jax.ShapeDtypeStruct((NQ, DA), jnp.bfloat16),
    )(x_query)


def _distances_tc(partial_sums, partial_counts, q_aug):
    def body(ps_ref, pc_ref, a_ref, out_ref, r_ref, pn_ref):
        @pl.when(pl.program_id(0) == 0)
        def _():
            sums_flat = ps_ref[0]
            for w in range(1, NW):
                sums_flat = sums_flat + ps_ref[w]
            sums = sums_flat.reshape(C, D)
            counts = jnp.sum(pc_ref[...], axis=0)
            proto = sums * (1.0 / counts)[:, None]
            pn_ref[...] = jnp.sum(proto * proto, axis=1)[None, :]
            lane = lax.broadcasted_iota(jnp.int32, (C, 128), 1)
            tail = jnp.where(lane < 2, -0.5, 0.0)
            r_ref[...] = jnp.concatenate(
                [proto.astype(jnp.bfloat16), tail.astype(jnp.bfloat16)],
                axis=1)

        dot = lax.dot_general(
            a_ref[...], r_ref[...], (((1,), (1,)), ((), ())),
            preferred_element_type=jnp.float32,
        )
        out_ref[...] = pn_ref[...] - 2.0 * dot

    return pl.pallas_call(
        body,
        grid=(NQ // QBLK,),
        in_specs=[
            pl.BlockSpec((NW, C * D), lambda i: (0, 0)),
            pl.BlockSpec((NW, C), lambda i: (0, 0)),
            pl.BlockSpec((QBLK, DA), lambda i: (i, 0)),
        ],
        out_specs=pl.BlockSpec((QBLK, C), lambda i: (i, 0)),
        out_shape=jax.ShapeDtypeStruct((NQ, C), jnp.float32),
        scratch_shapes=[
            pltpu.VMEM((C, DA), jnp.bfloat16),
            pltpu.VMEM((1, C), jnp.float32),
        ],
    )(partial_sums, partial_counts, q_aug)


def kernel(x_support, y_support, x_query):
    q_aug = _prep_queries_tc(x_query)
    sums, counts = _segment_sums_sc(
        x_support, y_support.astype(jnp.int32))
    return _distances_tc(sums, counts, q_aug)


# speedup vs baseline: 1.1066x; 1.0554x over previous
import dataclasses
import functools

import jax
import jax.numpy as jnp
from jax import lax
from jax.experimental import pallas as pl
from jax.experimental.pallas import tpu as pltpu
from jax.experimental.pallas import tpu_sc as plsc

C = 64
D = 512
NSUP = 8192
NQ = 16384
NC = 2
NT = 16
NW = NC * NT
ROWS_PER_TILE = NSUP // NW
CHUNK = 64
NCHUNK = ROWS_PER_TILE // CHUNK
QBLK = 4096


def _segment_sums_sc(x_support, y_support):
    mesh = plsc.VectorSubcoreMesh(core_axis_name="c", subcore_axis_name="s")
    cp = pltpu.CompilerParams()
    if "needs_layout_passes" in pltpu.CompilerParams.__dataclass_fields__:
        cp = dataclasses.replace(cp, needs_layout_passes=False)

    @functools.partial(
        pl.kernel,
        out_type=(
            jax.ShapeDtypeStruct((NW, C, D), jnp.float32),
            jax.ShapeDtypeStruct((NW, C), jnp.float32),
        ),
        mesh=mesh,
        scratch_types=[
            pltpu.VMEM((2, CHUNK, D), jnp.float32),
            pltpu.VMEM((2, CHUNK), jnp.int32),
            pltpu.VMEM((C, D), jnp.float32),
            pltpu.VMEM((C,), jnp.float32),
            pltpu.SemaphoreType.DMA,
            pltpu.SemaphoreType.DMA,
            pltpu.SemaphoreType.DMA,
            pltpu.SemaphoreType.DMA,
        ],
        compiler_params=cp,
    )
    def seg_kernel(x_hbm, y_hbm, sums_hbm, cnt_hbm, rows_v, idx_v, acc_v,
                   hist_v, semr0, semr1, semi0, semi1):
        cid = lax.axis_index("c")
        sid = lax.axis_index("s")
        wid = cid * NT + sid
        semr = (semr0, semr1)
        semi = (semi0, semi1)

        def start_fetch(k):
            b = k % 2
            off = wid * ROWS_PER_TILE + k * CHUNK
            hr = pltpu.async_copy(
                x_hbm.at[pl.ds(off, CHUNK)], rows_v.at[b], semr[b])
            hi = pltpu.async_copy(
                y_hbm.at[pl.ds(off, CHUNK)], idx_v.at[b], semi[b])
            return hr, hi

        inflight = start_fetch(0)

        zero16 = jnp.zeros((16,), jnp.float32)

        @pl.loop(0, C)
        def _(i):
            for u in range(D // 16):
                acc_v[i, pl.ds(u * 16, 16)] = zero16

        for u in range(C // 16):
            hist_v[pl.ds(u * 16, 16)] = zero16

        col0 = lax.broadcasted_iota(jnp.int32, (16,), 0)
        lane0 = col0 == 0
        ones16 = jnp.full((16,), 1.0, jnp.float32)
        zeroi16 = jnp.zeros((16,), jnp.int32)

        for k in range(NCHUNK):
            b = k % 2
            hr, hi = inflight
            hr.wait()
            hi.wait()
            if k + 1 < NCHUNK:
                inflight = start_fetch(k + 1)

            @plsc.parallel_loop(0, CHUNK, unroll=8)
            def _(r):
                lbl = plsc.load_gather(idx_v.at[b], [zeroi16 + r])
                plsc.addupdate_scatter(hist_v, [lbl], ones16, mask=lane0)
                for j in range(D // 16):
                    data = rows_v[b, r, pl.ds(j * 16, 16)]
                    plsc.addupdate_scatter(
                        acc_v, [lbl, col0 + j * 16], data)

        pltpu.sync_copy(acc_v, sums_hbm.at[wid])
        pltpu.sync_copy(hist_v, cnt_hbm.at[wid])

    return seg_kernel(x_support, y_support)


def _distances_tc(partial_sums, partial_counts, x_query):
    def body(ps_ref, pc_ref, q_ref, out_ref, proto_ref, pn_ref):
        @pl.when(pl.program_id(0) == 0)
        def _():
            sums = ps_ref[0]
            for w in range(1, NW):
                sums = sums + ps_ref[w]
            counts = jnp.sum(pc_ref[...], axis=0)
            proto = sums * (1.0 / counts)[:, None]
            proto_ref[...] = proto.astype(jnp.bfloat16)
            pn_ref[...] = jnp.sum(proto * proto, axis=1)[None, :]

        q = q_ref[...]
        qn = jnp.sum(q * q, axis=1, keepdims=True)
        dot = lax.dot_general(
            q.astype(jnp.bfloat16), proto_ref[...],
            (((1,), (1,)), ((), ())),
            preferred_element_type=jnp.float32,
        )
        out_ref[...] = qn - 2.0 * dot + pn_ref[...]

    return pl.pallas_call(
        body,
        grid=(NQ // QBLK,),
        in_specs=[
            pl.BlockSpec((NW, C, D), lambda i: (0, 0, 0)),
            pl.BlockSpec((NW, C), lambda i: (0, 0)),
            pl.BlockSpec((QBLK, D), lambda i: (i, 0)),
        ],
        out_specs=pl.BlockSpec((QBLK, C), lambda i: (i, 0)),
        out_shape=jax.ShapeDtypeStruct((NQ, C), jnp.float32),
        scratch_shapes=[
            pltpu.VMEM((C, D), jnp.bfloat16),
            pltpu.VMEM((1, C), jnp.float32),
        ],
    )(partial_sums, partial_counts, x_query)


def kernel(x_support, y_support, x_query):
    sums, counts = _segment_sums_sc(
        x_support, y_support.astype(jnp.int32))
    return _distances_tc(sums, counts, x_query)
